# baseline (speedup 1.0000x reference)
import jax
import jax.numpy as jnp
from jax import lax
from jax.experimental import pallas as pl
from jax.experimental.pallas import tpu as pltpu
from jax.experimental.pallas import tpu_sc as plsc

B = 16384
D = 64
UFD = 23
IFD = 19
FPAD = 32
PKW = 256
NW = 32
BPW = B // NW
NCH = BPW // 128


def _gather_body(ui_hbm, ii_hbm, ue_hbm, ie_hbm, ub_hbm, ib_hbm,
                 uft_hbm, ift_hbm,
                 pk_out, ub_out, ib_out,
                 ui_v, ii_v, fu_v, fi_v,
                 ue_v, ie_v, uf_v, if_v, ubs_v, ibs_v, sem):
    nc = 2
    wid = lax.axis_index("s") * nc + lax.axis_index("c")
    base = wid * BPW
    for r in range(NCH):
        pltpu.sync_copy(ui_hbm.at[pl.ds(base + r * 128, 128)], ui_v.at[r])
        pltpu.sync_copy(ii_hbm.at[pl.ds(base + r * 128, 128)], ii_v.at[r])

    nu = uft_hbm.shape[0]
    ni = ift_hbm.shape[0]

    for r in range(NCH):
        for c in range(8):
            u = ui_v[r, pl.ds(c * 16, 16)]
            i = ii_v[r, pl.ds(c * 16, 16)]
            fu_v[r, pl.ds(c * 16, 16)] = jnp.where(u == 0, nu - 1, u - 1)
            fi_v[r, pl.ds(c * 16, 16)] = jnp.where(i == 0, ni - 1, i - 1)

    cps = []
    for r in range(NCH):
        sl = pl.ds(r * 128, 128)
        cps += [
            pltpu.async_copy(ue_hbm.at[ui_v.at[r]], ue_v.at[sl], sem),
            pltpu.async_copy(ie_hbm.at[ii_v.at[r]], ie_v.at[sl], sem),
            pltpu.async_copy(uft_hbm.at[fu_v.at[r]], uf_v.at[sl], sem),
            pltpu.async_copy(ift_hbm.at[fi_v.at[r]], if_v.at[sl], sem),
            pltpu.async_copy(ub_hbm.at[ui_v.at[r]], ubs_v.at[sl], sem),
            pltpu.async_copy(ib_hbm.at[ii_v.at[r]], ibs_v.at[sl], sem),
        ]
    for cp in cps:
        cp.wait()

    rows = pl.ds(base, BPW)
    pltpu.sync_copy(ue_v, pk_out.at[rows, pl.ds(0, D)])
    pltpu.sync_copy(ie_v, pk_out.at[rows, pl.ds(D, D)])
    pltpu.sync_copy(uf_v, pk_out.at[rows, pl.ds(2 * D, FPAD)])
    pltpu.sync_copy(if_v, pk_out.at[rows, pl.ds(2 * D + FPAD, FPAD)])
    pltpu.sync_copy(ubs_v, ub_out.at[rows])
    pltpu.sync_copy(ibs_v, ib_out.at[rows])


def _sc_gather(ui, ii, ue, ie, ub1, ib1, uft24, ift24):
    mesh = plsc.VectorSubcoreMesh(core_axis_name="c", subcore_axis_name="s")
    f32 = jnp.float32
    i32 = jnp.int32
    bf16 = jnp.bfloat16
    out_type = (
        jax.ShapeDtypeStruct((B, PKW), bf16),
        jax.ShapeDtypeStruct((B,), f32),
        jax.ShapeDtypeStruct((B,), f32),
    )
    scratch = [
        pltpu.VMEM((NCH, 128), i32),
        pltpu.VMEM((NCH, 128), i32),
        pltpu.VMEM((NCH, 128), i32),
        pltpu.VMEM((NCH, 128), i32),
        pltpu.VMEM((BPW, D), bf16),
        pltpu.VMEM((BPW, D), bf16),
        pltpu.VMEM((BPW, FPAD), bf16),
        pltpu.VMEM((BPW, FPAD), bf16),
        pltpu.VMEM((BPW,), f32),
        pltpu.VMEM((BPW,), f32),
        pltpu.SemaphoreType.DMA,
    ]
    fn = pl.kernel(_gather_body, out_type=out_type, mesh=mesh,
                   scratch_types=scratch,
                   compiler_params=pltpu.CompilerParams(
                       use_tc_tiling_on_sc=False))
    return fn(ui, ii, ue, ie, ub1, ib1, uft24, ift24)


def _dense_body(pk_ref, ub_ref, ib_ref, wu_ref, wi_ref, out_ref):
    bs = ub_ref.shape[0]
    pk = pk_ref[...].reshape(bs, PKW).astype(jnp.float32)
    ue = pk[:, 0:D]
    ie = pk[:, D:2 * D]
    uf = pk[:, 2 * D:2 * D + FPAD]
    if_ = pk[:, 2 * D + FPAD:2 * D + 2 * FPAD]
    pu = lax.dot_general(uf, wu_ref[...], (((1,), (1,)), ((), ())),
                         preferred_element_type=jnp.float32)
    pi = lax.dot_general(if_, wi_ref[...], (((1,), (1,)), ((), ())),
                         preferred_element_type=jnp.float32)
    u = ue + jnp.maximum(pu, 0.0)
    i = ie + jnp.maximum(pi, 0.0)
    s = jnp.sum(u * i, axis=1) + ub_ref[...] + ib_ref[...]
    out_ref[...] = jax.nn.sigmoid(s) * 4.0 + 1.0


def _tc_dense(pk1, ub, ib, wu24, wi24):
    bs = 2048
    grid = (B // bs,)
    return pl.pallas_call(
        _dense_body,
        grid=grid,
        in_specs=[
            pl.BlockSpec((bs * PKW,), lambda i: (i,)),
            pl.BlockSpec((bs,), lambda i: (i,)),
            pl.BlockSpec((bs,), lambda i: (i,)),
            pl.BlockSpec((D, FPAD), lambda i: (0, 0)),
            pl.BlockSpec((D, FPAD), lambda i: (0, 0)),
        ],
        out_specs=pl.BlockSpec((bs,), lambda i: (i,)),
        out_shape=jax.ShapeDtypeStruct((B,), jnp.float32),
    )(pk1, ub, ib, wu24, wi24)


def kernel(user_indices, item_indices, user_emb, item_emb, user_bias,
           item_bias, user_feature, item_feature, W_user_feat, W_item_feat):
    bf16 = jnp.bfloat16
    ue16 = user_emb.astype(bf16)
    ie16 = item_emb.astype(bf16)
    uft24 = jnp.pad(user_feature.astype(bf16), ((0, 0), (0, FPAD - UFD)))
    ift24 = jnp.pad(item_feature.astype(bf16), ((0, 0), (0, FPAD - IFD)))
    wu24 = jnp.pad(W_user_feat, ((0, 0), (0, FPAD - UFD)))
    wi24 = jnp.pad(W_item_feat, ((0, 0), (0, FPAD - IFD)))
    ub1 = user_bias.reshape(-1)
    ib1 = item_bias.reshape(-1)
    pk, ub, ib = _sc_gather(user_indices, item_indices, ue16, ie16,
                            ub1, ib1, uft24, ift24)
    return _tc_dense(pk.reshape(-1), ub, ib, wu24, wi24)


# speedup vs baseline: 1.5150x; 1.5150x over previous
import jax
import jax.numpy as jnp
from jax import lax
from jax.experimental import pallas as pl
from jax.experimental.pallas import tpu as pltpu
from jax.experimental.pallas import tpu_sc as plsc

B = 16384
D = 64
UFD = 23
IFD = 19
FPAD = 24
PKW = 256
NW = 32
BPW = B // NW
NCH = BPW // 128


def _gather_body(ui_hbm, ii_hbm, ue_hbm, ie_hbm, ub_hbm, ib_hbm,
                 uft_hbm, ift_hbm,
                 pk_out, ub_out, ib_out,
                 ui_v, ii_v, fu_v, fi_v,
                 ue_v, ie_v, uf_v, if_v, ubs_v, ibs_v, sem):
    nc = 2
    wid = lax.axis_index("s") * nc + lax.axis_index("c")
    base = wid * BPW
    for r in range(NCH):
        pltpu.sync_copy(ui_hbm.at[pl.ds(base + r * 128, 128)], ui_v.at[r])
        pltpu.sync_copy(ii_hbm.at[pl.ds(base + r * 128, 128)], ii_v.at[r])

    nu = uft_hbm.shape[0]
    ni = ift_hbm.shape[0]

    for r in range(NCH):
        for c in range(8):
            u = ui_v[r, pl.ds(c * 16, 16)]
            i = ii_v[r, pl.ds(c * 16, 16)]
            fu_v[r, pl.ds(c * 16, 16)] = jnp.where(u == 0, nu - 1, u - 1)
            fi_v[r, pl.ds(c * 16, 16)] = jnp.where(i == 0, ni - 1, i - 1)

    cps = []
    for r in range(NCH):
        sl = pl.ds(r * 128, 128)
        cps += [
            pltpu.async_copy(ue_hbm.at[ui_v.at[r]], ue_v.at[sl], sem),
            pltpu.async_copy(ie_hbm.at[ii_v.at[r]], ie_v.at[sl], sem),
            pltpu.async_copy(uft_hbm.at[fu_v.at[r]], uf_v.at[sl], sem),
            pltpu.async_copy(ift_hbm.at[fi_v.at[r]], if_v.at[sl], sem),
            pltpu.async_copy(ub_hbm.at[ui_v.at[r]], ubs_v.at[sl], sem),
            pltpu.async_copy(ib_hbm.at[ii_v.at[r]], ibs_v.at[sl], sem),
        ]
    for cp in cps:
        cp.wait()

    rows = pl.ds(base, BPW)
    pltpu.sync_copy(ue_v, pk_out.at[rows, pl.ds(0, D)])
    pltpu.sync_copy(ie_v, pk_out.at[rows, pl.ds(D, D)])
    pltpu.sync_copy(uf_v, pk_out.at[rows, pl.ds(2 * D, FPAD)])
    pltpu.sync_copy(if_v, pk_out.at[rows, pl.ds(2 * D + FPAD, FPAD)])
    pltpu.sync_copy(ubs_v, ub_out.at[rows])
    pltpu.sync_copy(ibs_v, ib_out.at[rows])


def _sc_gather(ui, ii, ue, ie, ub1, ib1, uft24, ift24):
    mesh = plsc.VectorSubcoreMesh(core_axis_name="c", subcore_axis_name="s")
    f32 = jnp.float32
    i32 = jnp.int32
    out_type = (
        jax.ShapeDtypeStruct((B, PKW), f32),
        jax.ShapeDtypeStruct((B,), f32),
        jax.ShapeDtypeStruct((B,), f32),
    )
    scratch = [
        pltpu.VMEM((NCH, 128), i32),
        pltpu.VMEM((NCH, 128), i32),
        pltpu.VMEM((NCH, 128), i32),
        pltpu.VMEM((NCH, 128), i32),
        pltpu.VMEM((BPW, D), f32),
        pltpu.VMEM((BPW, D), f32),
        pltpu.VMEM((BPW, FPAD), f32),
        pltpu.VMEM((BPW, FPAD), f32),
        pltpu.VMEM((BPW,), f32),
        pltpu.VMEM((BPW,), f32),
        pltpu.SemaphoreType.DMA,
    ]
    fn = pl.kernel(_gather_body, out_type=out_type, mesh=mesh,
                   scratch_types=scratch,
                   compiler_params=pltpu.CompilerParams(
                       use_tc_tiling_on_sc=False))
    return fn(ui, ii, ue, ie, ub1, ib1, uft24, ift24)


def _dense_body(pk_ref, ub_ref, ib_ref, wu_ref, wi_ref, out_ref):
    bs = ub_ref.shape[0]
    pk = pk_ref[...].reshape(bs, PKW)
    ue = pk[:, 0:D]
    ie = pk[:, D:2 * D]
    uf = pk[:, 2 * D:2 * D + FPAD]
    if_ = pk[:, 2 * D + FPAD:2 * D + 2 * FPAD]
    pu = lax.dot_general(uf, wu_ref[...], (((1,), (1,)), ((), ())),
                         preferred_element_type=jnp.float32)
    pi = lax.dot_general(if_, wi_ref[...], (((1,), (1,)), ((), ())),
                         preferred_element_type=jnp.float32)
    u = ue + jnp.maximum(pu, 0.0)
    i = ie + jnp.maximum(pi, 0.0)
    s = jnp.sum(u * i, axis=1) + ub_ref[...] + ib_ref[...]
    out_ref[...] = jax.nn.sigmoid(s) * 4.0 + 1.0


def _tc_dense(pk1, ub, ib, wu24, wi24):
    bs = 2048
    grid = (B // bs,)
    return pl.pallas_call(
        _dense_body,
        grid=grid,
        in_specs=[
            pl.BlockSpec((bs * PKW,), lambda i: (i,)),
            pl.BlockSpec((bs,), lambda i: (i,)),
            pl.BlockSpec((bs,), lambda i: (i,)),
            pl.BlockSpec((D, FPAD), lambda i: (0, 0)),
            pl.BlockSpec((D, FPAD), lambda i: (0, 0)),
        ],
        out_specs=pl.BlockSpec((bs,), lambda i: (i,)),
        out_shape=jax.ShapeDtypeStruct((B,), jnp.float32),
    )(pk1, ub, ib, wu24, wi24)


def kernel(user_indices, item_indices, user_emb, item_emb, user_bias,
           item_bias, user_feature, item_feature, W_user_feat, W_item_feat):
    uft24 = jnp.pad(user_feature, ((0, 0), (0, FPAD - UFD)))
    ift24 = jnp.pad(item_feature, ((0, 0), (0, FPAD - IFD)))
    wu24 = jnp.pad(W_user_feat, ((0, 0), (0, FPAD - UFD)))
    wi24 = jnp.pad(W_item_feat, ((0, 0), (0, FPAD - IFD)))
    ub1 = user_bias.reshape(-1)
    ib1 = item_bias.reshape(-1)
    pk, ub, ib = _sc_gather(user_indices, item_indices, user_emb, item_emb,
                            ub1, ib1, uft24, ift24)
    return _tc_dense(pk.reshape(-1), ub, ib, wu24, wi24)
